# baseline (speedup 1.0000x reference)
import functools

import jax
import jax.numpy as jnp
from jax import lax
from jax.experimental import pallas as pl
from jax.experimental.pallas import tpu as pltpu
from jax.experimental.pallas import tpu_sc as plsc

N_NODES = 10000
N_EDGES = 320000
D = 128

NC = 2
NS = 16
NW = NC * NS
EPW = N_EDGES // NW
CHUNK = 80
NCHUNK = EPW // CHUNK
TOT_CHUNKS = N_EDGES // CHUNK
NBUF = 4
NPK = 6
PERIOD = 12
WB_CHUNK = 80
WB_GROUPS = N_NODES // WB_CHUNK
WB_ITERS = -(-WB_GROUPS // NS)
LANES = 16
DSTEPS = D // LANES


def _sc_body(x_hbm, packed_hbm, w_hbm, scal_hbm, out_hbm,
             pk0, pk1, pk2, pk3, pk4, pk5,
             wv0, wv1, wv2, wv3, r0, r1, r2, r3, scal_v, acc_sh,
             i0, i1, i2, i3, i4, i5,
             g0, g1, g2, g3, s0, s1, s2, s3, m0, m1, m2, m3):
    c = lax.axis_index("c")
    s = lax.axis_index("s")
    wid = s * NC + c
    base_cid = wid * NCHUNK

    pks = (pk0, pk1, pk2, pk3, pk4, pk5)
    wvs = (wv0, wv1, wv2, wv3)
    rows = (r0, r1, r2, r3)
    isems = (i0, i1, i2, i3, i4, i5)
    gsems = (g0, g1, g2, g3)
    ssems = (s0, s1, s2, s3)
    wsems = (m0, m1, m2, m3)

    pltpu.sync_copy(scal_hbm, scal_v)

    zero16 = jnp.zeros((LANES,), jnp.float32)

    def zero_row(i, _):
        for d in range(DSTEPS):
            r3[i, pl.ds(d * LANES, LANES)] = zero16
        return 0

    lax.fori_loop(0, WB_CHUNK, zero_row, 0)

    def zero_acc(t, _):
        g = s + t * NS

        @pl.when(g < WB_GROUPS)
        def _():
            pltpu.sync_copy(r3, acc_sh.at[pl.ds(g * WB_CHUNK, WB_CHUNK)])

        return 0

    lax.fori_loop(0, WB_ITERS, zero_acc, 0)
    plsc.subcore_barrier()

    scal_vec = scal_v[...]

    def fire_idx(p, t):
        pltpu.async_copy(packed_hbm.at[base_cid + t], pks[p], isems[p])

    def wait_idx(p, t):
        pltpu.make_async_copy(packed_hbm.at[base_cid + t], pks[p],
                              isems[p]).wait()

    def fire_gather(b, p, t):
        pltpu.async_copy(w_hbm.at[base_cid + t], wvs[b], wsems[b])
        pltpu.async_copy(x_hbm.at[pks[p].at[1]], rows[b], gsems[b])

    def wait_gather(b, p, t):
        pltpu.make_async_copy(w_hbm.at[base_cid + t], wvs[b],
                              wsems[b]).wait()
        pltpu.make_async_copy(x_hbm.at[pks[p].at[1]], rows[b],
                              gsems[b]).wait()

    def fire_scatter(b, p, t):
        pltpu.async_copy(rows[b], acc_sh.at[pks[p].at[0]], ssems[b],
                         add=True)

    def wait_scatter(b, p, t):
        pltpu.make_async_copy(rows[b], acc_sh.at[pks[p].at[0]],
                              ssems[b]).wait()

    def scale(b, t):
        def scale_grp(j, _):
            w16 = wvs[b][0, pl.ds(j * LANES, LANES)] * scal_vec
            base_e = j * LANES
            for lane in range(LANES):
                wsc = lax.broadcast_in_dim(w16[lane], (LANES,), ())
                e = base_e + lane
                for d in range(DSTEPS):
                    sl = pl.ds(d * LANES, LANES)
                    rows[b][e, sl] = rows[b][e, sl] * wsc
            return 0

        lax.fori_loop(0, CHUNK // LANES, scale_grp, 0)

    for tt in range(4):
        fire_idx(tt, tt)
    wait_idx(0, 0)
    fire_gather(0, 0, 0)
    wait_idx(1, 1)
    fire_gather(1, 1, 1)

    def step_block(t12, _):
        for off in range(PERIOD):
            t = t12 * PERIOD + off
            bb = off % NBUF
            pp = off % NPK

            @pl.when(t < NCHUNK)
            def _():
                wait_gather(bb, pp, t)
                scale(bb, t)
                fire_scatter(bb, pp, t)
                b2 = (bb + 2) % NBUF
                p2 = (pp + 2) % NPK
                p4 = (pp + 4) % NPK

                @pl.when(t + 2 < NCHUNK)
                def _():
                    @pl.when(t >= 2)
                    def _():
                        wait_scatter(b2, p4, t - 2)

                    @pl.when(t + 4 < NCHUNK)
                    def _():
                        fire_idx(p4, t + 4)

                    wait_idx(p2, t + 2)
                    fire_gather(b2, p2, t + 2)

        return 0

    lax.fori_loop(0, -(-NCHUNK // PERIOD), step_block, 0)

    for tt in range(NCHUNK - NBUF, NCHUNK):
        wait_scatter(tt % NBUF, tt % NPK, tt)

    plsc.subcore_barrier()

    def writeback(t, _):
        g = s + t * NS

        @pl.when(g < WB_GROUPS)
        def _():
            off = g * WB_CHUNK
            pltpu.sync_copy(acc_sh.at[pl.ds(off, WB_CHUNK)],
                            out_hbm.at[c, pl.ds(off, WB_CHUNK)])

        return 0

    lax.fori_loop(0, WB_ITERS, writeback, 0)


_sc_kernel = functools.partial(
    pl.kernel,
    out_type=jax.ShapeDtypeStruct((NC, N_NODES, D), jnp.float32),
    mesh=plsc.VectorSubcoreMesh(core_axis_name="c", subcore_axis_name="s"),
    scratch_types=[
        pltpu.VMEM((2, CHUNK), jnp.int32),
        pltpu.VMEM((2, CHUNK), jnp.int32),
        pltpu.VMEM((2, CHUNK), jnp.int32),
        pltpu.VMEM((2, CHUNK), jnp.int32),
        pltpu.VMEM((2, CHUNK), jnp.int32),
        pltpu.VMEM((2, CHUNK), jnp.int32),
        pltpu.VMEM((1, CHUNK), jnp.float32),
        pltpu.VMEM((1, CHUNK), jnp.float32),
        pltpu.VMEM((1, CHUNK), jnp.float32),
        pltpu.VMEM((1, CHUNK), jnp.float32),
        pltpu.VMEM((CHUNK, D), jnp.float32),
        pltpu.VMEM((CHUNK, D), jnp.float32),
        pltpu.VMEM((CHUNK, D), jnp.float32),
        pltpu.VMEM((CHUNK, D), jnp.float32),
        pltpu.VMEM((LANES,), jnp.float32),
        pltpu.VMEM_SHARED((N_NODES, D), jnp.float32),
        pltpu.SemaphoreType.DMA,
        pltpu.SemaphoreType.DMA,
        pltpu.SemaphoreType.DMA,
        pltpu.SemaphoreType.DMA,
        pltpu.SemaphoreType.DMA,
        pltpu.SemaphoreType.DMA,
        pltpu.SemaphoreType.DMA,
        pltpu.SemaphoreType.DMA,
        pltpu.SemaphoreType.DMA,
        pltpu.SemaphoreType.DMA,
        pltpu.SemaphoreType.DMA,
        pltpu.SemaphoreType.DMA,
        pltpu.SemaphoreType.DMA,
        pltpu.SemaphoreType.DMA,
        pltpu.SemaphoreType.DMA,
        pltpu.SemaphoreType.DMA,
        pltpu.SemaphoreType.DMA,
        pltpu.SemaphoreType.DMA,
    ],
)(_sc_body)


_TC_ROWS = 1000


def _combine_body(p_ref, o_ref):
    a = p_ref[0] + p_ref[1]
    o_ref[...] = jnp.where(a > 0, a, jnp.exp(a) - 1.0)


_combine = pl.pallas_call(
    _combine_body,
    grid=(N_NODES // _TC_ROWS,),
    in_specs=[pl.BlockSpec((NC, _TC_ROWS, D), lambda i: (0, i, 0))],
    out_specs=pl.BlockSpec((_TC_ROWS, D), lambda i: (i, 0)),
    out_shape=jax.ShapeDtypeStruct((N_NODES, D), jnp.float32),
)


def kernel(x, edge_index, edge_weight, scalar):
    row = edge_index[0].astype(jnp.int32)
    col = edge_index[1].astype(jnp.int32)
    packed = jnp.stack(
        [row.reshape(TOT_CHUNKS, CHUNK),
         col.reshape(TOT_CHUNKS, CHUNK)], axis=1)
    w = edge_weight.astype(jnp.float32).reshape(TOT_CHUNKS, 1, CHUNK)
    scal16 = jnp.broadcast_to(scalar.astype(jnp.float32), (LANES,))
    partial = _sc_kernel(x, packed, w, scal16)
    return _combine(partial)
